# Initial kernel scaffold; baseline (speedup 1.0000x reference)
import jax
import jax.numpy as jnp
from jax.experimental import pallas as pl


def kernel(x, edge_index, W1, b1, W2, b2):
    raise NotImplementedError("write your pallas kernel here")


# speedup vs baseline: 6.6608x; 6.6608x over previous
import functools

import jax
import jax.numpy as jnp
from jax import lax
from jax.experimental import pallas as pl
from jax.experimental.pallas import tpu as pltpu
from jax.experimental.pallas import tpu_sc as plsc

N = 10000
E = 320000
F = 128
CH = 128
NCHUNK = E // CH
NC = 2
NS = 16
NW = NC * NS
RPT = N // NS
ZR = 125


def _make_agg():
    mesh = plsc.VectorSubcoreMesh(core_axis_name="c", subcore_axis_name="s")

    @functools.partial(
        pl.kernel,
        mesh=mesh,
        out_type=jax.ShapeDtypeStruct((NC, N, F), jnp.float32),
        scratch_types=[
            pltpu.VMEM((CH,), jnp.int32),
            pltpu.VMEM((CH,), jnp.int32),
            pltpu.VMEM((CH, F), jnp.float32),
            pltpu.VMEM((ZR, F), jnp.float32),
            pltpu.VMEM_SHARED((N, F), jnp.float32),
            pltpu.SemaphoreType.DMA,
        ],
    )
    def agg(y_hbm, src_hbm, dst_hbm, part_hbm, sidx, didx, rows, zbuf, acc, sem):
        c = lax.axis_index("c")
        s = lax.axis_index("s")
        wid = s * NC + c

        def zstore(k, carry):
            i = k // (F // 16)
            j = (k % (F // 16)) * 16
            zbuf[i, pl.ds(j, 16)] = jnp.zeros((16,), jnp.float32)
            return carry

        lax.fori_loop(0, ZR * (F // 16), zstore, 0)

        def zcopy(i, carry):
            pltpu.sync_copy(zbuf, acc.at[pl.ds(s * RPT + i * ZR, ZR)])
            return carry

        lax.fori_loop(0, RPT // ZR, zcopy, 0)
        plsc.subcore_barrier()

        nch = (NCHUNK // NW) + jnp.where(wid < NCHUNK % NW, 1, 0)

        def body(i, carry):
            e0 = (wid + NW * i) * CH
            pltpu.sync_copy(src_hbm.at[pl.ds(e0, CH)], sidx)
            pltpu.sync_copy(dst_hbm.at[pl.ds(e0, CH)], didx)
            pltpu.async_copy(y_hbm.at[sidx], rows, sem).wait()
            pltpu.sync_copy(rows, acc.at[didx], add=True)
            return carry

        lax.fori_loop(0, nch, body, 0)
        plsc.subcore_barrier()

        r0 = pl.multiple_of(s * 632, 8)

        @pl.when(s < NS - 1)
        def _flush_main():
            pltpu.sync_copy(acc.at[pl.ds(r0, 632)],
                            part_hbm.at[c, pl.ds(r0, 632)])

        @pl.when(s == NS - 1)
        def _flush_tail():
            pltpu.sync_copy(acc.at[pl.ds(15 * 632, 520)],
                            part_hbm.at[c, pl.ds(15 * 632, 520)])

    return agg


_AGG = None


def _get_agg():
    global _AGG
    if _AGG is None:
        _AGG = _make_agg()
    return _AGG


BM = 1000


def _matmul(x, w):
    m, k = x.shape
    n = w.shape[1]

    def body(x_ref, w_ref, o_ref):
        o_ref[...] = jnp.dot(x_ref[...], w_ref[...],
                             preferred_element_type=jnp.float32)

    return pl.pallas_call(
        body,
        grid=(m // BM,),
        in_specs=[pl.BlockSpec((BM, k), lambda i: (i, 0)),
                  pl.BlockSpec((k, n), lambda i: (0, 0))],
        out_specs=pl.BlockSpec((BM, n), lambda i: (i, 0)),
        out_shape=jax.ShapeDtypeStruct((m, n), jnp.float32),
    )(x, w)


def _mid(y, parts, b, w):
    m, n = y.shape

    def body(y_ref, p_ref, b_ref, w_ref, o_ref):
        h = y_ref[...] + p_ref[0] + p_ref[1] + b_ref[...]
        h = jnp.maximum(h, 0.0)
        o_ref[...] = jnp.dot(h, w_ref[...], preferred_element_type=jnp.float32)

    return pl.pallas_call(
        body,
        grid=(m // BM,),
        in_specs=[pl.BlockSpec((BM, n), lambda i: (i, 0)),
                  pl.BlockSpec((NC, BM, n), lambda i: (0, i, 0)),
                  pl.BlockSpec((1, n), lambda i: (0, 0)),
                  pl.BlockSpec((n, n), lambda i: (0, 0))],
        out_specs=pl.BlockSpec((BM, n), lambda i: (i, 0)),
        out_shape=jax.ShapeDtypeStruct((m, n), jnp.float32),
    )(y, parts, b.reshape(1, n), w)


def _final(y, parts, b):
    m, n = y.shape

    def body(y_ref, p_ref, b_ref, o_ref):
        o_ref[...] = jnp.maximum(
            y_ref[...] + p_ref[0] + p_ref[1] + b_ref[...], 0.0)

    return pl.pallas_call(
        body,
        grid=(m // BM,),
        in_specs=[pl.BlockSpec((BM, n), lambda i: (i, 0)),
                  pl.BlockSpec((NC, BM, n), lambda i: (0, i, 0)),
                  pl.BlockSpec((1, n), lambda i: (0, 0))],
        out_specs=pl.BlockSpec((BM, n), lambda i: (i, 0)),
        out_shape=jax.ShapeDtypeStruct((m, n), jnp.float32),
    )(y, parts, b.reshape(1, n))


def kernel(x, edge_index, W1, b1, W2, b2):
    src = edge_index[0]
    dst = edge_index[1]
    agg = _get_agg()
    y1 = _matmul(x, W1)
    p1 = agg(y1, src, dst)
    y2 = _mid(y1, p1, b1, W2)
    p2 = agg(y2, src, dst)
    return _final(y2, p2, b2)
